# baseline (speedup 1.0000x reference)
import functools

import jax
import jax.numpy as jnp
from jax import lax
from jax.experimental import pallas as pl
from jax.experimental.pallas import tpu as pltpu
from jax.experimental.pallas import tpu_sc as plsc

E = 16
K = 2
TM = 128
NEG = -1e30


def _gate_body(x_ref, gw_ref, gb_ref, i1_ref, i2_ref, s1_ref, s2_ref,
               xb_ref):
    x = x_ref[...].astype(jnp.bfloat16)
    n_, d_ = x.shape
    xb_ref[...] = pltpu.bitcast(x.reshape(2 * n_, d_ // 2), jnp.float32)
    logits = jax.lax.dot_general(
        x, gw_ref[...].astype(jnp.bfloat16), (((1,), (0,)), ((), ())),
        preferred_element_type=jnp.float32) + gb_ref[...][None, :]
    n = logits.shape[0]
    lane = jax.lax.broadcasted_iota(jnp.int32, (n, E), 1)
    m1 = jnp.max(logits, axis=1, keepdims=True)
    i1 = jnp.min(jnp.where(logits == m1, lane, E), axis=1, keepdims=True)
    masked = jnp.where(lane == i1, NEG, logits)
    m2 = jnp.max(masked, axis=1, keepdims=True)
    i2 = jnp.min(jnp.where(masked == m2, lane, E), axis=1, keepdims=True)
    e = jnp.exp(m2 - m1)
    s1 = 1.0 / (1.0 + e)
    i1_ref[...] = i1
    i2_ref[...] = i2
    s1_ref[...] = s1
    s2_ref[...] = e * s1


def _gate(x, gw, gb):
    n, d = x.shape
    return pl.pallas_call(
        _gate_body,
        out_shape=[
            jax.ShapeDtypeStruct((n, 1), jnp.int32),
            jax.ShapeDtypeStruct((n, 1), jnp.int32),
            jax.ShapeDtypeStruct((n, 1), jnp.float32),
            jax.ShapeDtypeStruct((n, 1), jnp.float32),
            jax.ShapeDtypeStruct((n, d // 2), jnp.float32),
        ],
    )(x, gw, gb)


_RB = 512


def _rank_body(idx_ref, rank_ref):
    blk = pl.program_id(0)
    m = idx_ref.shape[0]
    lane = jax.lax.broadcasted_iota(jnp.int32, (m, E), 1)
    oh = (idx_ref[...] == lane).astype(jnp.bfloat16)
    row_g = jax.lax.broadcasted_iota(jnp.int32, (_RB, m), 0) + blk * _RB
    col = jax.lax.broadcasted_iota(jnp.int32, (_RB, m), 1)
    tri = (col <= row_g).astype(jnp.bfloat16)
    cum = jax.lax.dot_general(
        tri, oh, (((1,), (0,)), ((), ())),
        preferred_element_type=jnp.float32)
    lane_b = jax.lax.broadcasted_iota(jnp.int32, (_RB, E), 1)
    oh_blk = (idx_ref[pl.ds(blk * _RB, _RB), :] == lane_b).astype(jnp.float32)
    rank = jnp.sum(cum * oh_blk, axis=1, keepdims=True) - 1.0
    rank_ref[...] = rank.astype(jnp.int32)


def _rank(idx_flat):
    m = idx_flat.shape[0]
    return pl.pallas_call(
        _rank_body,
        grid=(m // _RB,),
        in_specs=[pl.BlockSpec((m, 1), lambda b: (0, 0))],
        out_specs=pl.BlockSpec((_RB, 1), lambda b: (b, 0)),
        out_shape=jax.ShapeDtypeStruct((m, 1), jnp.int32),
    )(idx_flat)


def _meta_body(idx_ref, rank_ref, dest_ref, xb_ref, eb_ref, nt_ref, t_max):
    m = idx_ref.shape[0]
    lane = jax.lax.broadcasted_iota(jnp.int32, (m, E), 1)
    oh = (idx_ref[...] == lane).astype(jnp.float32)
    counts = jnp.sum(oh, axis=0, keepdims=True)
    padded = jnp.floor((counts + (TM - 1)) / TM) * TM
    r16 = jax.lax.broadcasted_iota(jnp.int32, (E, E), 0)
    c16 = jax.lax.broadcasted_iota(jnp.int32, (E, E), 1)
    stri = (c16 < r16).astype(jnp.float32)
    off = jax.lax.dot_general(
        stri, padded.reshape(E, 1), (((1,), (0,)), ((), ())),
        precision=jax.lax.Precision.HIGHEST,
        preferred_element_type=jnp.float32).reshape(1, E)
    dest = rank_ref[...].astype(jnp.float32) + jnp.sum(
        oh * off, axis=1, keepdims=True)
    dest_ref[...] = dest.astype(jnp.int32)
    offtile = (off / TM).astype(jnp.int32)
    n_tiles = jnp.sum(padded).astype(jnp.int32) // TM
    nt_ref[...] = jnp.full((1, 1), 1, jnp.int32) * n_tiles
    t_iota = jax.lax.broadcasted_iota(jnp.int32, (t_max, 1), 0)
    xb = jnp.minimum(t_iota, n_tiles - 1)
    xb_ref[...] = xb
    eb_ref[...] = jnp.sum(
        (offtile <= xb).astype(jnp.int32), axis=1, keepdims=True) - 1


def _meta(idx_flat, rank, t_max):
    m = idx_flat.shape[0]
    return pl.pallas_call(
        functools.partial(_meta_body, t_max=t_max),
        out_shape=[
            jax.ShapeDtypeStruct((m, 1), jnp.int32),
            jax.ShapeDtypeStruct((t_max, 1), jnp.int32),
            jax.ShapeDtypeStruct((t_max, 1), jnp.int32),
            jax.ShapeDtypeStruct((1, 1), jnp.int32),
        ],
    )(idx_flat, rank)


def _ffn_body(xb_ref, eb_ref, nt_ref, x_ref, w1_ref, b1_ref, w2_ref,
              b2_ref, y_ref):
    t = pl.program_id(0)

    @pl.when(t < nt_ref[0])
    def _():
        tm, d2 = x_ref.shape
        x16 = pltpu.bitcast(x_ref[...], jnp.bfloat16).reshape(tm, 2 * d2)
        h = jax.lax.dot_general(
            x16, w1_ref[0], (((1,), (0,)), ((), ())),
            preferred_element_type=jnp.float32) + b1_ref[0]
        h = jax.nn.gelu(h)
        y = jax.lax.dot_general(
            h, w2_ref[0], (((1,), (0,)), ((), ())),
            preferred_element_type=jnp.float32) + b2_ref[0]
        y_ref[...] = pltpu.bitcast(
            y.astype(jnp.bfloat16).reshape(2 * tm, d2), jnp.float32)


def _ffn(xb, eb, nt, xs, w1, b1, w2, b2, t_max):
    d2 = xs.shape[1]
    d = w1.shape[1]
    dff = w1.shape[2]
    grid_spec = pltpu.PrefetchScalarGridSpec(
        num_scalar_prefetch=3,
        grid=(t_max,),
        in_specs=[
            pl.BlockSpec((TM, d2), lambda t, xb, eb, nt: (xb[t], 0)),
            pl.BlockSpec((1, d, dff), lambda t, xb, eb, nt: (eb[t], 0, 0)),
            pl.BlockSpec((1, 1, dff), lambda t, xb, eb, nt: (eb[t], 0, 0)),
            pl.BlockSpec((1, dff, d), lambda t, xb, eb, nt: (eb[t], 0, 0)),
            pl.BlockSpec((1, 1, d), lambda t, xb, eb, nt: (eb[t], 0, 0)),
        ],
        out_specs=pl.BlockSpec((TM, d2), lambda t, xb, eb, nt: (xb[t], 0)),
    )
    return pl.pallas_call(
        _ffn_body,
        grid_spec=grid_spec,
        out_shape=jax.ShapeDtypeStruct((t_max * TM, d2), jnp.float32),
    )(xb, eb, nt, xs, w1, b1.reshape(E, 1, dff), w2, b2.reshape(E, 1, d))


_NW = 32
_CH = 64


def _dispatch(x, pos1, pos2, pad_rows):
    n, d = x.shape
    mesh = plsc.VectorSubcoreMesh(core_axis_name="c", subcore_axis_name="s")

    @functools.partial(
        pl.kernel,
        out_type=jax.ShapeDtypeStruct((pad_rows, d), jnp.float32),
        mesh=mesh,
        scratch_types=[
            pltpu.VMEM((_CH, d), jnp.float32),
            pltpu.VMEM((_CH,), jnp.int32),
            pltpu.VMEM((_CH,), jnp.int32),
            pltpu.SemaphoreType.DMA,
        ],
    )
    def run(x_hbm, p1_hbm, p2_hbm, xs_hbm, xv, i1v, i2v, sem):
        wid = lax.axis_index("s") * 2 + lax.axis_index("c")
        base = wid * _CH
        pltpu.sync_copy(x_hbm.at[pl.ds(base, _CH)], xv)
        pltpu.sync_copy(p1_hbm.at[pl.ds(base, _CH)], i1v)
        pltpu.sync_copy(p2_hbm.at[pl.ds(base, _CH)], i2v)
        c1 = pltpu.async_copy(xv, xs_hbm.at[i1v], sem)
        c2 = pltpu.async_copy(xv, xs_hbm.at[i2v], sem)
        c1.wait()
        c2.wait()

    return run(x, pos1, pos2)


def _gather2(ys, pos1, pos2):
    n = pos1.shape[0]
    d = ys.shape[1]
    mesh = plsc.VectorSubcoreMesh(core_axis_name="c", subcore_axis_name="s")

    @functools.partial(
        pl.kernel,
        out_type=[jax.ShapeDtypeStruct((n, d), jnp.float32),
                  jax.ShapeDtypeStruct((n, d), jnp.float32)],
        mesh=mesh,
        scratch_types=[
            pltpu.VMEM((_CH, d), jnp.float32),
            pltpu.VMEM((_CH, d), jnp.float32),
            pltpu.VMEM((_CH,), jnp.int32),
            pltpu.VMEM((_CH,), jnp.int32),
            pltpu.SemaphoreType.DMA,
            pltpu.SemaphoreType.DMA,
        ],
    )
    def run(ys_hbm, p1_hbm, p2_hbm, r1_hbm, r2_hbm, rv1, rv2, iv1, iv2,
            sem1, sem2):
        wid = lax.axis_index("s") * 2 + lax.axis_index("c")
        base = wid * _CH
        pltpu.sync_copy(p1_hbm.at[pl.ds(base, _CH)], iv1)
        pltpu.sync_copy(p2_hbm.at[pl.ds(base, _CH)], iv2)
        c1 = pltpu.async_copy(ys_hbm.at[iv1], rv1, sem1)
        c2 = pltpu.async_copy(ys_hbm.at[iv2], rv2, sem2)
        c1.wait()
        c2.wait()
        pltpu.sync_copy(rv1, r1_hbm.at[pl.ds(base, _CH)])
        pltpu.sync_copy(rv2, r2_hbm.at[pl.ds(base, _CH)])

    return run(ys, pos1, pos2)


def _combine_body(r1_ref, r2_ref, s1_ref, s2_ref, o_ref):
    blk, d2 = r1_ref.shape
    r1 = pltpu.bitcast(
        r1_ref[...], jnp.bfloat16).reshape(blk, 2 * d2).astype(jnp.float32)
    r2 = pltpu.bitcast(
        r2_ref[...], jnp.bfloat16).reshape(blk, 2 * d2).astype(jnp.float32)
    o_ref[...] = s1_ref[...] * r1 + s2_ref[...] * r2


def _combine(r1, r2, s1, s2):
    n, d2 = r1.shape
    blk = 256
    return pl.pallas_call(
        _combine_body,
        grid=(n // blk,),
        in_specs=[
            pl.BlockSpec((blk, d2), lambda b: (b, 0)),
            pl.BlockSpec((blk, d2), lambda b: (b, 0)),
            pl.BlockSpec((blk, 1), lambda b: (b, 0)),
            pl.BlockSpec((blk, 1), lambda b: (b, 0)),
        ],
        out_specs=pl.BlockSpec((blk, 2 * d2), lambda b: (b, 0)),
        out_shape=jax.ShapeDtypeStruct((n, 2 * d2), jnp.float32),
    )(r1, r2, s1, s2)


def kernel(moe_inp, gate_w, gate_b, w1, b1, w2, b2):
    n, d = moe_inp.shape
    t_max = (n * K) // TM + E - 1

    i1, i2, s1, s2, x16 = _gate(moe_inp, gate_w, gate_b)
    idx_flat = jnp.concatenate([i1, i2], axis=0)
    rank = _rank(idx_flat)
    dest, xb, eb, nt = _meta(idx_flat, rank, t_max)
    pos1 = dest[:n, 0]
    pos2 = dest[n:, 0]

    xs = _dispatch(x16, pos1, pos2, t_max * TM)
    ys = _ffn(xb[:, 0], eb[:, 0], nt[:, 0], xs, w1, b1, w2, b2, t_max)
    r1, r2 = _gather2(ys, pos1, pos2)
    return _combine(r1, r2, s1, s2)


# speedup vs baseline: 1.1936x; 1.1936x over previous
import functools

import jax
import jax.numpy as jnp
from jax import lax
from jax.experimental import pallas as pl
from jax.experimental.pallas import tpu as pltpu
from jax.experimental.pallas import tpu_sc as plsc

E = 16
K = 2
TM = 256
NEG = -1e30


def _gm_body(x_ref, gw_ref, gb_ref, s1_ref, s2_ref, dest_ref, xb_ref,
             eb_ref, nt_ref, xb16_ref, t_max):
    x = x_ref[...].astype(jnp.bfloat16)
    n, d = x.shape
    xb16_ref[...] = pltpu.bitcast(x.reshape(2 * n, d // 2), jnp.float32)
    logits = jax.lax.dot_general(
        x, gw_ref[...].astype(jnp.bfloat16), (((1,), (0,)), ((), ())),
        preferred_element_type=jnp.float32) + gb_ref[...][None, :]
    lane = jax.lax.broadcasted_iota(jnp.int32, (n, E), 1)
    m1 = jnp.max(logits, axis=1, keepdims=True)
    i1 = jnp.min(jnp.where(logits == m1, lane, E), axis=1, keepdims=True)
    masked = jnp.where(lane == i1, NEG, logits)
    m2 = jnp.max(masked, axis=1, keepdims=True)
    i2 = jnp.min(jnp.where(masked == m2, lane, E), axis=1, keepdims=True)
    e = jnp.exp(m2 - m1)
    s1 = 1.0 / (1.0 + e)
    s1_ref[...] = s1
    s2_ref[...] = e * s1

    idx_flat = jnp.concatenate([i1, i2], axis=0)
    m = 2 * n
    rb = 512
    r_io = jax.lax.broadcasted_iota(jnp.int32, (rb, rb), 0)
    c_io = jax.lax.broadcasted_iota(jnp.int32, (rb, rb), 1)
    tri = (c_io <= r_io).astype(jnp.bfloat16)
    carry = jnp.zeros((1, E), jnp.float32)
    ranks = []
    for b in range(m // rb):
        idx_b = jax.lax.slice(idx_flat, (b * rb, 0), ((b + 1) * rb, 1))
        lane_b = jax.lax.broadcasted_iota(jnp.int32, (rb, E), 1)
        oh_b = (idx_b == lane_b).astype(jnp.bfloat16)
        cum_b = jax.lax.dot_general(
            tri, oh_b, (((1,), (0,)), ((), ())),
            preferred_element_type=jnp.float32) + carry
        oh_f = oh_b.astype(jnp.float32)
        ranks.append(jnp.sum(cum_b * oh_f, axis=1, keepdims=True) - 1.0)
        carry = carry + jnp.sum(oh_f, axis=0, keepdims=True)
    rank = jnp.concatenate(ranks, axis=0)

    counts = carry
    padded = jnp.floor((counts + (TM - 1)) / TM) * TM
    r16 = jax.lax.broadcasted_iota(jnp.int32, (E, E), 0)
    c16 = jax.lax.broadcasted_iota(jnp.int32, (E, E), 1)
    stri = (c16 < r16).astype(jnp.float32)
    off = jax.lax.dot_general(
        stri, padded.reshape(E, 1), (((1,), (0,)), ((), ())),
        precision=jax.lax.Precision.HIGHEST,
        preferred_element_type=jnp.float32).reshape(1, E)
    lane_m = jax.lax.broadcasted_iota(jnp.int32, (m, E), 1)
    oh = (idx_flat == lane_m).astype(jnp.float32)
    dest = rank + jnp.sum(oh * off, axis=1, keepdims=True)
    dest_ref[...] = dest.astype(jnp.int32)
    offtile = (off / TM).astype(jnp.int32)
    n_tiles = jnp.sum(padded).astype(jnp.int32) // TM
    nt_ref[...] = jnp.full((1, 1), 1, jnp.int32) * n_tiles
    t_iota = jax.lax.broadcasted_iota(jnp.int32, (t_max, 1), 0)
    xb = jnp.minimum(t_iota, n_tiles - 1)
    xb_ref[...] = xb
    eb_ref[...] = jnp.sum(
        (offtile <= xb).astype(jnp.int32), axis=1, keepdims=True) - 1


def _gate_meta(x, gw, gb, t_max):
    n, d = x.shape
    return pl.pallas_call(
        functools.partial(_gm_body, t_max=t_max),
        out_shape=[
            jax.ShapeDtypeStruct((n, 1), jnp.float32),
            jax.ShapeDtypeStruct((n, 1), jnp.float32),
            jax.ShapeDtypeStruct((2 * n, 1), jnp.int32),
            jax.ShapeDtypeStruct((t_max, 1), jnp.int32),
            jax.ShapeDtypeStruct((t_max, 1), jnp.int32),
            jax.ShapeDtypeStruct((1, 1), jnp.int32),
            jax.ShapeDtypeStruct((n, d // 2), jnp.float32),
        ],
    )(x, gw, gb)


_NW = 32
_CH = 64


def _dispatch(x, pos1, pos2, pad_rows):
    n, d = x.shape
    mesh = plsc.VectorSubcoreMesh(core_axis_name="c", subcore_axis_name="s")

    @functools.partial(
        pl.kernel,
        out_type=jax.ShapeDtypeStruct((pad_rows, d), jnp.float32),
        mesh=mesh,
        scratch_types=[
            pltpu.VMEM((_CH, d), jnp.float32),
            pltpu.VMEM((_CH,), jnp.int32),
            pltpu.VMEM((_CH,), jnp.int32),
            pltpu.SemaphoreType.DMA,
        ],
    )
    def run(x_hbm, p1_hbm, p2_hbm, xs_hbm, xv, i1v, i2v, sem):
        wid = lax.axis_index("s") * 2 + lax.axis_index("c")
        base = wid * _CH
        pltpu.sync_copy(x_hbm.at[pl.ds(base, _CH)], xv)
        pltpu.sync_copy(p1_hbm.at[pl.ds(base, _CH)], i1v)
        pltpu.sync_copy(p2_hbm.at[pl.ds(base, _CH)], i2v)
        c1 = pltpu.async_copy(xv, xs_hbm.at[i1v], sem)
        c2 = pltpu.async_copy(xv, xs_hbm.at[i2v], sem)
        c1.wait()
        c2.wait()

    return run(x, pos1, pos2)


def _ffn_body(xb_ref, eb_ref, nt_ref, x_ref, w1_ref, b1_ref, w2_ref,
              b2_ref, y_ref):
    t = pl.program_id(0)

    @pl.when(t < nt_ref[0])
    def _():
        tm, d2 = x_ref.shape
        x16 = pltpu.bitcast(x_ref[...], jnp.bfloat16).reshape(tm, 2 * d2)
        h = jax.lax.dot_general(
            x16, w1_ref[0], (((1,), (0,)), ((), ())),
            preferred_element_type=jnp.float32) + b1_ref[0]
        h = jax.nn.gelu(h)
        y = jax.lax.dot_general(
            h, w2_ref[0], (((1,), (0,)), ((), ())),
            preferred_element_type=jnp.float32) + b2_ref[0]
        y_ref[...] = pltpu.bitcast(
            y.astype(jnp.bfloat16).reshape(2 * tm, d2), jnp.float32)


def _ffn(xb, eb, nt, xs, w1, b1, w2, b2, t_max):
    d2 = xs.shape[1]
    d = w1.shape[1]
    dff = w1.shape[2]
    grid_spec = pltpu.PrefetchScalarGridSpec(
        num_scalar_prefetch=3,
        grid=(t_max,),
        in_specs=[
            pl.BlockSpec((TM, d2), lambda t, xb, eb, nt: (xb[t], 0)),
            pl.BlockSpec((1, d, dff), lambda t, xb, eb, nt: (eb[t], 0, 0)),
            pl.BlockSpec((1, 1, dff), lambda t, xb, eb, nt: (eb[t], 0, 0)),
            pl.BlockSpec((1, dff, d), lambda t, xb, eb, nt: (eb[t], 0, 0)),
            pl.BlockSpec((1, 1, d), lambda t, xb, eb, nt: (eb[t], 0, 0)),
        ],
        out_specs=pl.BlockSpec((TM, d2), lambda t, xb, eb, nt: (xb[t], 0)),
    )
    return pl.pallas_call(
        _ffn_body,
        grid_spec=grid_spec,
        out_shape=jax.ShapeDtypeStruct((t_max * TM, d2), jnp.float32),
    )(xb, eb, nt, xs, w1, b1.reshape(E, 1, dff), w2, b2.reshape(E, 1, d))


def _gather2(ys, pos1, pos2):
    n = pos1.shape[0]
    d = ys.shape[1]
    mesh = plsc.VectorSubcoreMesh(core_axis_name="c", subcore_axis_name="s")

    @functools.partial(
        pl.kernel,
        out_type=[jax.ShapeDtypeStruct((n, d), jnp.float32),
                  jax.ShapeDtypeStruct((n, d), jnp.float32)],
        mesh=mesh,
        scratch_types=[
            pltpu.VMEM((_CH, d), jnp.float32),
            pltpu.VMEM((_CH, d), jnp.float32),
            pltpu.VMEM((_CH,), jnp.int32),
            pltpu.VMEM((_CH,), jnp.int32),
            pltpu.SemaphoreType.DMA,
            pltpu.SemaphoreType.DMA,
        ],
    )
    def run(ys_hbm, p1_hbm, p2_hbm, r1_hbm, r2_hbm, rv1, rv2, iv1, iv2,
            sem1, sem2):
        wid = lax.axis_index("s") * 2 + lax.axis_index("c")
        base = wid * _CH
        pltpu.sync_copy(p1_hbm.at[pl.ds(base, _CH)], iv1)
        pltpu.sync_copy(p2_hbm.at[pl.ds(base, _CH)], iv2)
        c1 = pltpu.async_copy(ys_hbm.at[iv1], rv1, sem1)
        c2 = pltpu.async_copy(ys_hbm.at[iv2], rv2, sem2)
        c1.wait()
        c2.wait()
        pltpu.sync_copy(rv1, r1_hbm.at[pl.ds(base, _CH)])
        pltpu.sync_copy(rv2, r2_hbm.at[pl.ds(base, _CH)])

    return run(ys, pos1, pos2)


def _combine_body(r1_ref, r2_ref, s1_ref, s2_ref, o_ref):
    blk, d2 = r1_ref.shape
    r1 = pltpu.bitcast(
        r1_ref[...], jnp.bfloat16).reshape(blk, 2 * d2).astype(jnp.float32)
    r2 = pltpu.bitcast(
        r2_ref[...], jnp.bfloat16).reshape(blk, 2 * d2).astype(jnp.float32)
    o_ref[...] = s1_ref[...] * r1 + s2_ref[...] * r2


def _combine(r1, r2, s1, s2):
    n, d2 = r1.shape
    blk = 256
    return pl.pallas_call(
        _combine_body,
        grid=(n // blk,),
        in_specs=[
            pl.BlockSpec((blk, d2), lambda b: (b, 0)),
            pl.BlockSpec((blk, d2), lambda b: (b, 0)),
            pl.BlockSpec((blk, 1), lambda b: (b, 0)),
            pl.BlockSpec((blk, 1), lambda b: (b, 0)),
        ],
        out_specs=pl.BlockSpec((blk, 2 * d2), lambda b: (b, 0)),
        out_shape=jax.ShapeDtypeStruct((n, 2 * d2), jnp.float32),
    )(r1, r2, s1, s2)


def kernel(moe_inp, gate_w, gate_b, w1, b1, w2, b2):
    n, d = moe_inp.shape
    t_max = (n * K) // TM + E - 1

    s1, s2, dest, xb, eb, nt, x16 = _gate_meta(moe_inp, gate_w, gate_b,
                                               t_max)
    pos1 = dest[:n, 0]
    pos2 = dest[n:, 0]

    xs = _dispatch(x16, pos1, pos2, t_max * TM)
    ys = _ffn(xb[:, 0], eb[:, 0], nt[:, 0], xs, w1, b1, w2, b2, t_max)
    r1, r2 = _gather2(ys, pos1, pos2)
    return _combine(r1, r2, s1, s2)
